# baseline (speedup 1.0000x reference)
import functools

import jax
import jax.numpy as jnp
from jax import lax
from jax.experimental import pallas as pl
from jax.experimental.pallas import tpu as pltpu
from jax.experimental.pallas import tpu_sc as plsc

NC = 2
NS = 16
NW = NC * NS
CHUNK = 128
SUP = 16


def _sc_agg(x, src2, dst2, batch, attr, zs, zc, ones):
    n_src, d = x.shape
    n_chunks = src2.shape[0]
    chunks_per_tile = n_chunks // NW
    n_sup = chunks_per_tile // SUP
    n_dst = zs.shape[0]
    stripe = 256
    s_tail = n_dst - (NS - 1) * stripe
    a_tail = n_dst - (NW - 1) * 128

    mesh = plsc.VectorSubcoreMesh(core_axis_name="c", subcore_axis_name="s")

    @functools.partial(
        pl.kernel,
        out_type=(
            jax.ShapeDtypeStruct((NC, n_dst, d), jnp.float32),
            jax.ShapeDtypeStruct((NC, n_dst, d), jnp.float32),
            jax.ShapeDtypeStruct((n_dst, d), jnp.float32),
        ),
        mesh=mesh,
        scratch_types=(
            pltpu.VMEM((SUP, CHUNK), jnp.int32),
            pltpu.VMEM((SUP, CHUNK), jnp.int32),
            pltpu.VMEM((CHUNK, d), jnp.float32),
            pltpu.VMEM((CHUNK, d), jnp.float32),
            pltpu.VMEM((CHUNK, d), jnp.float32),
            pltpu.SemaphoreType.DMA,
            pltpu.SemaphoreType.DMA,
            pltpu.SemaphoreType.DMA,
            pltpu.SemaphoreType.DMA,
            pltpu.VMEM_SHARED((n_dst, d), jnp.float32),
            pltpu.VMEM_SHARED((n_dst, d), jnp.float32),
        ),
    )
    def body(x_h, src_h, dst_h, batch_h, attr_h, zs_h, zc_h, ones_h,
             summed_o, cnt_o, attr_o,
             src_v, dst_v, rows0, rows1, ones_v,
             g0, g1, s0, s1, summed_sh, cnt_sh):
        c = lax.axis_index("c")
        s = lax.axis_index("s")
        wid = s * NC + c
        rows = (rows0, rows1)
        gsem = (g0, g1)
        ssem = (s0, s1)

        r0 = pl.multiple_of(s * stripe, stripe)

        @pl.when(s < NS - 1)
        def _zero_full():
            pltpu.sync_copy(zs_h.at[pl.ds(r0, stripe)],
                            summed_sh.at[pl.ds(r0, stripe)])
            pltpu.sync_copy(zc_h.at[pl.ds(r0, stripe)],
                            cnt_sh.at[pl.ds(r0, stripe)])

        @pl.when(s == NS - 1)
        def _zero_tail():
            t0 = (NS - 1) * stripe
            pltpu.sync_copy(zs_h.at[pl.ds(t0, s_tail)],
                            summed_sh.at[pl.ds(t0, s_tail)])
            pltpu.sync_copy(zc_h.at[pl.ds(t0, s_tail)],
                            cnt_sh.at[pl.ds(t0, s_tail)])

        pltpu.sync_copy(ones_h, ones_v)
        plsc.subcore_barrier()

        def sup_body(k, carry):
            cbase = pl.multiple_of((wid * n_sup + k) * SUP, 8)
            pltpu.sync_copy(src_h.at[pl.ds(cbase, SUP)], src_v)
            pltpu.sync_copy(dst_h.at[pl.ds(cbase, SUP)], dst_v)
            pltpu.async_copy(x_h.at[src_v.at[0]], rows0, g0)
            pltpu.async_copy(x_h.at[src_v.at[1]], rows1, g1)
            for j in range(SUP):
                b = j % 2
                pltpu.make_async_copy(x_h.at[src_v.at[j]], rows[b],
                                      gsem[b]).wait()
                cp_r = pltpu.async_copy(rows[b], summed_sh.at[dst_v.at[j]],
                                        ssem[b], add=True)
                cp_o = pltpu.async_copy(ones_v, cnt_sh.at[dst_v.at[j]],
                                        ssem[b], add=True)
                cp_r.wait()
                cp_o.wait()
                if j + 2 < SUP:
                    pltpu.async_copy(x_h.at[src_v.at[j + 2]], rows[b],
                                     gsem[b])
            return carry

        lax.fori_loop(0, n_sup, sup_body, 0)
        plsc.subcore_barrier()

        @pl.when(s < NS - 1)
        def _pub_full():
            pltpu.sync_copy(summed_sh.at[pl.ds(r0, stripe)],
                            summed_o.at[c, pl.ds(r0, stripe)])
            pltpu.sync_copy(cnt_sh.at[pl.ds(r0, stripe)],
                            cnt_o.at[c, pl.ds(r0, stripe)])

        @pl.when(s == NS - 1)
        def _pub_tail():
            t0 = (NS - 1) * stripe
            pltpu.sync_copy(summed_sh.at[pl.ds(t0, s_tail)],
                            summed_o.at[c, pl.ds(t0, s_tail)])
            pltpu.sync_copy(cnt_sh.at[pl.ds(t0, s_tail)],
                            cnt_o.at[c, pl.ds(t0, s_tail)])

        @pl.when(wid < NW - 1)
        def _full():
            b = pl.multiple_of(wid * 128, 128)
            pltpu.sync_copy(batch_h.at[pl.ds(b, 128)], src_v.at[0])
            pltpu.async_copy(attr_h.at[src_v.at[0]], rows0, g0).wait()
            pltpu.sync_copy(rows0, attr_o.at[pl.ds(b, 128)])

        @pl.when(wid == NW - 1)
        def _tail():
            b = (NW - 1) * 128
            pltpu.sync_copy(batch_h.at[pl.ds(b, a_tail)],
                            src_v.at[0, pl.ds(0, a_tail)])
            pltpu.async_copy(attr_h.at[src_v.at[0, pl.ds(0, a_tail)]],
                             rows0.at[pl.ds(0, a_tail)], g0).wait()
            pltpu.sync_copy(rows0.at[pl.ds(0, a_tail)],
                            attr_o.at[pl.ds(b, a_tail)])

    return body(x, src2, dst2, batch, attr, zs, zc, ones)


def _tc_combine(summed2, cnt2, x_t, attr_g, W_l, W_r, W_lin, b_l, b_lin,
                n_pad):
    n_dst, d = x_t.shape
    blk = 1000
    grid = n_dst // blk
    dn = (((1,), (1,)), ((), ()))
    fpad = float(n_pad)

    def body(s2, c2, xt, ag, wl, wr, wlin, bl, blin, o):
        ssum = s2[0] + s2[1]
        cnt = c2[0] + c2[1]
        row = lax.broadcasted_iota(jnp.int32, (blk, 1), 0)
        corr = jnp.where((row == 0) & (pl.program_id(0) == 0), fpad, 0.0)
        ssum = ssum - corr * xt[0:1, :]
        mean = ssum / jnp.maximum(cnt[:, 0:1] - corr, 1.0)
        acc = lax.dot_general(mean, wl[...], dn,
                              preferred_element_type=jnp.float32)
        acc = acc + lax.dot_general(xt[...], wr[...], dn,
                                    preferred_element_type=jnp.float32)
        acc = acc + 0.25 * lax.dot_general(ag[...], wlin[...], dn,
                                           preferred_element_type=jnp.float32)
        acc = acc + (bl[...] + 0.25 * blin[...])
        o[...] = jnp.maximum(acc, 0.0)

    return pl.pallas_call(
        body,
        grid=(grid,),
        in_specs=[
            pl.BlockSpec((NC, blk, d), lambda i: (0, i, 0)),
            pl.BlockSpec((NC, blk, d), lambda i: (0, i, 0)),
            pl.BlockSpec((blk, d), lambda i: (i, 0)),
            pl.BlockSpec((blk, d), lambda i: (i, 0)),
            pl.BlockSpec((d, d), lambda i: (0, 0)),
            pl.BlockSpec((d, d), lambda i: (0, 0)),
            pl.BlockSpec((d, d), lambda i: (0, 0)),
            pl.BlockSpec((1, d), lambda i: (0, 0)),
            pl.BlockSpec((1, d), lambda i: (0, 0)),
        ],
        out_specs=pl.BlockSpec((blk, d), lambda i: (i, 0)),
        out_shape=jax.ShapeDtypeStruct((n_dst, d), jnp.float32),
    )(summed2, cnt2, x_t, attr_g, W_l, W_r, W_lin, b_l, b_lin)


def kernel(x, edge_index, batch, attr, W_l, b_l, W_r, W_lin, b_lin,
           size_src, size_dst):
    src = edge_index[0]
    dst = edge_index[1]
    n_dst = batch.shape[0]
    e = src.shape[0]
    step = NW * CHUNK * 8
    e_pad = -(-e // step) * step
    n_pad = e_pad - e
    pad = jnp.zeros((n_pad,), jnp.int32)
    src2 = jnp.concatenate([src, pad]).reshape(e_pad // CHUNK, CHUNK)
    dst2 = jnp.concatenate([dst, pad]).reshape(e_pad // CHUNK, CHUNK)
    zs = jnp.zeros((n_dst, x.shape[1]), jnp.float32)
    zc = jnp.zeros((n_dst, x.shape[1]), jnp.float32)
    ones = jnp.ones((CHUNK, x.shape[1]), jnp.float32)
    summed2, cnt2, attr_g = _sc_agg(x, src2, dst2, batch, attr, zs, zc, ones)
    return _tc_combine(summed2, cnt2, x[:n_dst], attr_g, W_l, W_r, W_lin,
                       b_l.reshape(1, -1), b_lin.reshape(1, -1), n_pad)


# speedup vs baseline: 1.0013x; 1.0013x over previous
import functools

import jax
import jax.numpy as jnp
from jax import lax
from jax.experimental import pallas as pl
from jax.experimental.pallas import tpu as pltpu
from jax.experimental.pallas import tpu_sc as plsc

NC = 2
NS = 16
NW = NC * NS
CHUNK = 128
SUP = 16


def _sc_agg(x, src2, dst2, batch, attr, zs, zc, ones):
    n_src, d = x.shape
    n_chunks = src2.shape[0]
    chunks_per_tile = n_chunks // NW
    n_sup = chunks_per_tile // SUP
    n_dst = zs.shape[0]
    stripe = 256
    s_tail = n_dst - (NS - 1) * stripe
    a_tail = n_dst - (NW - 1) * 128

    mesh = plsc.VectorSubcoreMesh(core_axis_name="c", subcore_axis_name="s")

    @functools.partial(
        pl.kernel,
        out_type=(
            jax.ShapeDtypeStruct((NC, n_dst, d), jnp.float32),
            jax.ShapeDtypeStruct((NC, n_dst, d), jnp.float32),
            jax.ShapeDtypeStruct((n_dst, d), jnp.float32),
        ),
        mesh=mesh,
        scratch_types=(
            pltpu.VMEM((SUP, CHUNK), jnp.int32),
            pltpu.VMEM((SUP, CHUNK), jnp.int32),
            pltpu.VMEM((CHUNK, d), jnp.float32),
            pltpu.VMEM((CHUNK, d), jnp.float32),
            pltpu.VMEM((CHUNK, d), jnp.float32),
            pltpu.SemaphoreType.DMA,
            pltpu.SemaphoreType.DMA,
            pltpu.SemaphoreType.DMA,
            pltpu.SemaphoreType.DMA,
            pltpu.VMEM_SHARED((n_dst, d), jnp.float32),
            pltpu.VMEM_SHARED((n_dst, d), jnp.float32),
        ),
    )
    def body(x_h, src_h, dst_h, batch_h, attr_h, zs_h, zc_h, ones_h,
             summed_o, cnt_o, attr_o,
             src_v, dst_v, rows0, rows1, ones_v,
             g0, g1, s0, s1, summed_sh, cnt_sh):
        c = lax.axis_index("c")
        s = lax.axis_index("s")
        wid = s * NC + c
        rows = (rows0, rows1)
        gsem = (g0, g1)
        ssem = (s0, s1)

        r0 = pl.multiple_of(s * stripe, stripe)

        @pl.when(s < NS - 1)
        def _zero_full():
            pltpu.sync_copy(zs_h.at[pl.ds(r0, stripe)],
                            summed_sh.at[pl.ds(r0, stripe)])
            pltpu.sync_copy(zc_h.at[pl.ds(r0, stripe)],
                            cnt_sh.at[pl.ds(r0, stripe)])

        @pl.when(s == NS - 1)
        def _zero_tail():
            t0 = (NS - 1) * stripe
            pltpu.sync_copy(zs_h.at[pl.ds(t0, s_tail)],
                            summed_sh.at[pl.ds(t0, s_tail)])
            pltpu.sync_copy(zc_h.at[pl.ds(t0, s_tail)],
                            cnt_sh.at[pl.ds(t0, s_tail)])

        pltpu.sync_copy(ones_h, ones_v)
        plsc.subcore_barrier()

        def sup_body(k, carry):
            cbase = pl.multiple_of((wid * n_sup + k) * SUP, 8)
            pltpu.sync_copy(src_h.at[pl.ds(cbase, SUP)], src_v)
            pltpu.sync_copy(dst_h.at[pl.ds(cbase, SUP)], dst_v)
            pltpu.async_copy(x_h.at[src_v.at[0]], rows0, g0)
            pltpu.async_copy(x_h.at[src_v.at[1]], rows1, g1)
            for j in range(SUP):
                b = j % 2
                pltpu.make_async_copy(x_h.at[src_v.at[j]], rows[b],
                                      gsem[b]).wait()
                pltpu.sync_copy(rows[b], summed_sh.at[dst_v.at[j]], add=True)
                pltpu.sync_copy(ones_v, cnt_sh.at[dst_v.at[j]], add=True)
                if j + 2 < SUP:
                    pltpu.async_copy(x_h.at[src_v.at[j + 2]], rows[b],
                                     gsem[b])
            return carry

        lax.fori_loop(0, n_sup, sup_body, 0)
        plsc.subcore_barrier()

        @pl.when(s < NS - 1)
        def _pub_full():
            pltpu.sync_copy(summed_sh.at[pl.ds(r0, stripe)],
                            summed_o.at[c, pl.ds(r0, stripe)])
            pltpu.sync_copy(cnt_sh.at[pl.ds(r0, stripe)],
                            cnt_o.at[c, pl.ds(r0, stripe)])

        @pl.when(s == NS - 1)
        def _pub_tail():
            t0 = (NS - 1) * stripe
            pltpu.sync_copy(summed_sh.at[pl.ds(t0, s_tail)],
                            summed_o.at[c, pl.ds(t0, s_tail)])
            pltpu.sync_copy(cnt_sh.at[pl.ds(t0, s_tail)],
                            cnt_o.at[c, pl.ds(t0, s_tail)])

        @pl.when(wid < NW - 1)
        def _full():
            b = pl.multiple_of(wid * 128, 128)
            pltpu.sync_copy(batch_h.at[pl.ds(b, 128)], src_v.at[0])
            pltpu.async_copy(attr_h.at[src_v.at[0]], rows0, g0).wait()
            pltpu.sync_copy(rows0, attr_o.at[pl.ds(b, 128)])

        @pl.when(wid == NW - 1)
        def _tail():
            b = (NW - 1) * 128
            pltpu.sync_copy(batch_h.at[pl.ds(b, a_tail)],
                            src_v.at[0, pl.ds(0, a_tail)])
            pltpu.async_copy(attr_h.at[src_v.at[0, pl.ds(0, a_tail)]],
                             rows0.at[pl.ds(0, a_tail)], g0).wait()
            pltpu.sync_copy(rows0.at[pl.ds(0, a_tail)],
                            attr_o.at[pl.ds(b, a_tail)])

    return body(x, src2, dst2, batch, attr, zs, zc, ones)


def _tc_combine(summed2, cnt2, x_t, attr_g, W_l, W_r, W_lin, b_l, b_lin,
                n_pad):
    n_dst, d = x_t.shape
    blk = 1000
    grid = n_dst // blk
    dn = (((1,), (1,)), ((), ()))
    fpad = float(n_pad)

    def body(s2, c2, xt, ag, wl, wr, wlin, bl, blin, o):
        ssum = s2[0] + s2[1]
        cnt = c2[0] + c2[1]
        row = lax.broadcasted_iota(jnp.int32, (blk, 1), 0)
        corr = jnp.where((row == 0) & (pl.program_id(0) == 0), fpad, 0.0)
        ssum = ssum - corr * xt[0:1, :]
        mean = ssum / jnp.maximum(cnt[:, 0:1] - corr, 1.0)
        acc = lax.dot_general(mean, wl[...], dn,
                              preferred_element_type=jnp.float32)
        acc = acc + lax.dot_general(xt[...], wr[...], dn,
                                    preferred_element_type=jnp.float32)
        acc = acc + 0.25 * lax.dot_general(ag[...], wlin[...], dn,
                                           preferred_element_type=jnp.float32)
        acc = acc + (bl[...] + 0.25 * blin[...])
        o[...] = jnp.maximum(acc, 0.0)

    return pl.pallas_call(
        body,
        grid=(grid,),
        in_specs=[
            pl.BlockSpec((NC, blk, d), lambda i: (0, i, 0)),
            pl.BlockSpec((NC, blk, d), lambda i: (0, i, 0)),
            pl.BlockSpec((blk, d), lambda i: (i, 0)),
            pl.BlockSpec((blk, d), lambda i: (i, 0)),
            pl.BlockSpec((d, d), lambda i: (0, 0)),
            pl.BlockSpec((d, d), lambda i: (0, 0)),
            pl.BlockSpec((d, d), lambda i: (0, 0)),
            pl.BlockSpec((1, d), lambda i: (0, 0)),
            pl.BlockSpec((1, d), lambda i: (0, 0)),
        ],
        out_specs=pl.BlockSpec((blk, d), lambda i: (i, 0)),
        out_shape=jax.ShapeDtypeStruct((n_dst, d), jnp.float32),
    )(summed2, cnt2, x_t, attr_g, W_l, W_r, W_lin, b_l, b_lin)


def kernel(x, edge_index, batch, attr, W_l, b_l, W_r, W_lin, b_lin,
           size_src, size_dst):
    src = edge_index[0]
    dst = edge_index[1]
    n_dst = batch.shape[0]
    e = src.shape[0]
    step = NW * CHUNK * 8
    e_pad = -(-e // step) * step
    n_pad = e_pad - e
    pad = jnp.zeros((n_pad,), jnp.int32)
    src2 = jnp.concatenate([src, pad]).reshape(e_pad // CHUNK, CHUNK)
    dst2 = jnp.concatenate([dst, pad]).reshape(e_pad // CHUNK, CHUNK)
    zs = jnp.zeros((n_dst, x.shape[1]), jnp.float32)
    zc = jnp.zeros((n_dst, x.shape[1]), jnp.float32)
    ones = jnp.ones((CHUNK, x.shape[1]), jnp.float32)
    summed2, cnt2, attr_g = _sc_agg(x, src2, dst2, batch, attr, zs, zc, ones)
    return _tc_combine(summed2, cnt2, x[:n_dst], attr_g, W_l, W_r, W_lin,
                       b_l.reshape(1, -1), b_lin.reshape(1, -1), n_pad)
